# baseline (speedup 1.0000x reference)
import functools

import jax
import jax.numpy as jnp
from jax import lax
from jax.experimental import pallas as pl
from jax.experimental.pallas import tpu as pltpu

BIG = 1e10
SENT = 2e5
ROWS = 32


def _chamfer_body(bins_smem, y_ref, out_ref, A_ref, chamy_ref, cnt_ref,
                  *, L, B, P, n_chunks):
    A_ref[...] = jnp.full((B, L * P, 8, 128), BIG, jnp.float32)
    chamy_ref[...] = jnp.zeros((L, B, ROWS, 128), jnp.float32)
    cnt_ref[...] = jnp.zeros((B, ROWS, 128), jnp.float32)

    for b in range(B):
        def chunk_body(i, _):
            yv = y_ref[b, i]
            mask = yv >= 0.001
            ys = jnp.where(mask, yv, SENT)
            cnt_ref[b] += mask.astype(jnp.float32)
            ys3 = ys.reshape(ROWS // 8, 8, 128)
            for l in range(L):
                def p_body(p, miny):
                    c = 0.5 * (bins_smem[l, b, p] + bins_smem[l, b, p + 1])
                    d = ys3 - c
                    d2 = d * d
                    miny = jnp.minimum(miny, d2)
                    t = jnp.min(d2, axis=0)
                    A_ref[b, l * P + p] = jnp.minimum(A_ref[b, l * P + p], t)
                    return miny
                miny = lax.fori_loop(
                    0, P, p_body,
                    jnp.full((ROWS // 8, 8, 128), jnp.float32(1e30)),
                    unroll=2)
                chamy_ref[l, b] += jnp.where(
                    mask, miny.reshape(ROWS, 128), jnp.float32(0.0))
            return 0
        lax.fori_loop(0, n_chunks, chunk_body, 0)

    loss = jnp.float32(0.0)
    for b in range(B):
        length = jnp.sum(cnt_ref[b])
        for l in range(L):
            s = jnp.float32(0.0)
            for j0 in range(0, P, 32):
                slab = A_ref[b, pl.ds(l * P + j0, 32)]
                m = jnp.min(slab, axis=1)
                s = s + jnp.sum(jnp.min(m, axis=1, keepdims=True))
            cham_x = s / jnp.float32(P)
            cham_y = jnp.sum(chamy_ref[l, b]) / length
            loss = loss + cham_x + cham_y
    out_ref[0, 0] = loss / jnp.float32(B)


def kernel(bins, target_depth_maps):
    L, B, P1 = bins.shape
    P = P1 - 1
    Bt = target_depth_maps.shape[0]
    M = target_depth_maps.size // Bt
    n_chunks = M // (ROWS * 128)
    assert M == n_chunks * ROWS * 128
    y = target_depth_maps.reshape(Bt, n_chunks, ROWS, 128)

    body = functools.partial(_chamfer_body, L=L, B=Bt, P=P, n_chunks=n_chunks)
    out = pl.pallas_call(
        body,
        in_specs=[
            pl.BlockSpec(memory_space=pltpu.SMEM),
            pl.BlockSpec(memory_space=pltpu.VMEM),
        ],
        out_specs=pl.BlockSpec(memory_space=pltpu.SMEM),
        out_shape=jax.ShapeDtypeStruct((1, 1), jnp.float32),
        scratch_shapes=[
            pltpu.VMEM((Bt, L * P, 8, 128), jnp.float32),
            pltpu.VMEM((L, Bt, ROWS, 128), jnp.float32),
            pltpu.VMEM((Bt, ROWS, 128), jnp.float32),
        ],
    )(bins, y)
    return out[0, 0]


# speedup vs baseline: 1.0982x; 1.0982x over previous
import functools

import jax
import jax.numpy as jnp
from jax import lax
from jax.experimental import pallas as pl
from jax.experimental.pallas import tpu as pltpu
from jax.experimental.pallas import tpu_sc as plsc

L = 4
B = 4
P = 128
E = P + 1
M = 49152
NW = 32
SPB = 8
MS = M // SPB
NV = MS // 16
CT = 256
RST = 144
IROW = L * RST
SROW = 128
BIGD = 1e10


def _cmpx(a, b):
    return jnp.minimum(a, b), jnp.maximum(a, b)


def _bclean(xs):
    if len(xs) == 1:
        return [jnp.sort(xs[0])]
    h = len(xs) // 2
    los, his = [], []
    for i in range(h):
        lo, hi = _cmpx(xs[i], xs[i + h])
        los.append(lo)
        his.append(hi)
    return _bclean(los) + _bclean(his)


def _msort(vs):
    if len(vs) == 1:
        return [jnp.sort(vs[0])]
    h = len(vs) // 2
    a = _msort(vs[:h])
    b = _msort(vs[h:])
    return _bclean(a + [jnp.flip(v, 0) for v in reversed(b)])


def _k1_body(bins_hbm, y_hbm, imax_o, imin_o, ctab_o, sums_o,
             bins_v, ctab_v, y_v, ys_v, rank_v, imax_v, imin_v, sums_v):
    cid = lax.axis_index("c")
    sid = lax.axis_index("s")
    w = sid * 2 + cid
    b = w // SPB
    sl = w % SPB

    pltpu.sync_copy(bins_hbm, bins_v)
    pltpu.sync_copy(y_hbm.at[pl.ds(b * M + sl * MS, MS)], y_v)

    iota = lax.iota(jnp.int32, 16)
    big16 = jnp.full((16,), 1e30, jnp.float32)
    for k in range(L * CT // 16):
        ctab_v[pl.ds(16 * k, 16)] = big16

    for l in range(L):
        base = l * (B * E) + b * E
        vs = []
        for k in range(8):
            idx = iota + (base + 16 * k)
            e0 = plsc.load_gather(bins_v, [idx])
            e1 = plsc.load_gather(bins_v, [idx + 1])
            vs.append(0.5 * (e0 + e1))
        svs = _msort(vs)
        for k in range(8):
            ctab_v[pl.ds(l * CT + 16 * k, 16)] = svs[k]

    neg16 = jnp.full((16,), -2e5, jnp.float32)
    pos16 = jnp.full((16,), 2e5, jnp.float32)
    for k in range(IROW // 16):
        imax_v[pl.ds(16 * k, 16)] = neg16
        imin_v[pl.ds(16 * k, 16)] = pos16

    idx_m1 = jnp.maximum(iota - 1, 0)
    idx_p1 = jnp.minimum(iota + 1, 15)
    lane0 = iota == 0
    lane15 = iota == 15

    def body(i, carry):
        s0, s1, s2, s3, cntv = carry
        yv = y_v[pl.ds(i * 16, 16)]
        yso = jnp.sort(yv)
        valid = yso >= 0.001
        cntv = cntv + jnp.where(valid, 1.0, 0.0)
        ys_v[...] = yso
        prev_y = plsc.load_gather(ys_v, [idx_m1])
        prev_valid = prev_y >= 0.001
        sums = [s0, s1, s2, s3]
        new_sums = []
        for l in range(L):
            cbase = l * CT
            pos0 = jnp.full((16,), cbase - 1, jnp.int32)
            for step in (128, 64, 32, 16, 8, 4, 2, 1):
                cand = pos0 + step
                cv = plsc.load_gather(ctab_v, [cand])
                pos0 = jnp.where(cv <= yso, cand, pos0)
            rank = pos0 - (cbase - 1)
            lo = plsc.load_gather(ctab_v, [jnp.maximum(pos0, cbase)])
            hi = plsc.load_gather(ctab_v, [jnp.minimum(pos0 + 1, cbase + P - 1)])
            dlo = yso - lo
            dhi = hi - yso
            dmin = jnp.minimum(dlo * dlo, dhi * dhi)
            new_sums.append(sums[l] + jnp.where(valid, dmin, 0.0))
            rank_v[...] = rank
            prev_rank = plsc.load_gather(rank_v, [idx_m1])
            next_rank = plsc.load_gather(rank_v, [idx_p1])
            is_last = valid & (lane15 | (next_rank != rank))
            is_first = valid & (lane0 | (prev_rank != rank)
                                | jnp.logical_not(prev_valid))
            ridx = rank + l * RST
            curmax = plsc.load_gather(imax_v, [ridx])
            plsc.store_scatter(imax_v, [ridx], jnp.maximum(curmax, yso),
                               mask=is_last)
            curmin = plsc.load_gather(imin_v, [ridx])
            plsc.store_scatter(imin_v, [ridx], jnp.minimum(curmin, yso),
                               mask=is_first)
        return (*new_sums, cntv)

    z16 = jnp.zeros((16,), jnp.float32)
    s0, s1, s2, s3, cntv = lax.fori_loop(
        0, NV, body, (z16, z16, z16, z16, z16))

    for l, sv in enumerate((s0, s1, s2, s3)):
        sums_v[pl.ds(l * 16, 16)] = sv
    sums_v[pl.ds(4 * 16, 16)] = cntv
    for k in range(5, 8):
        sums_v[pl.ds(k * 16, 16)] = z16

    pltpu.sync_copy(imax_v, imax_o.at[pl.ds(w * IROW, IROW)])
    pltpu.sync_copy(imin_v, imin_o.at[pl.ds(w * IROW, IROW)])
    pltpu.sync_copy(sums_v, sums_o.at[pl.ds(w * SROW, SROW)])

    @pl.when(sl == 0)
    def _():
        pltpu.sync_copy(ctab_v, ctab_o.at[pl.ds(b * (L * CT), L * CT)])


def _k2_body(imax_h, imin_h, ctab_h, sums_h, out_h,
             gmax_v, gmin_v, tmp_v, pm_v, sm_v, c_v, t16_v, out_v):
    cid = lax.axis_index("c")
    sid = lax.axis_index("s")
    w = sid * 2 + cid

    @pl.when(w < L * B)
    def _():
        l = w // B
        b = w % B
        iota = lax.iota(jnp.int32, 16)
        base = l * RST

        pltpu.sync_copy(imax_h.at[pl.ds((b * SPB) * IROW + base, RST)], gmax_v)
        pltpu.sync_copy(imin_h.at[pl.ds((b * SPB) * IROW + base, RST)], gmin_v)
        for s in range(1, SPB):
            off = (b * SPB + s) * IROW + base
            pltpu.sync_copy(imax_h.at[pl.ds(off, RST)], tmp_v)
            for k in range(RST // 16):
                gmax_v[pl.ds(16 * k, 16)] = jnp.maximum(
                    gmax_v[pl.ds(16 * k, 16)], tmp_v[pl.ds(16 * k, 16)])
            pltpu.sync_copy(imin_h.at[pl.ds(off, RST)], tmp_v)
            for k in range(RST // 16):
                gmin_v[pl.ds(16 * k, 16)] = jnp.minimum(
                    gmin_v[pl.ds(16 * k, 16)], tmp_v[pl.ds(16 * k, 16)])

        carry = jnp.float32(-3e5)
        for k in range(RST // 16):
            v = gmax_v[pl.ds(16 * k, 16)]
            cm = jnp.maximum(plsc.cummax(v), carry)
            pm_v[pl.ds(16 * k, 16)] = cm
            carry = jnp.max(cm)
        carry2 = jnp.float32(3e5)
        for k in range(RST // 16 - 1, -1, -1):
            v = gmin_v[pl.ds(16 * k, 16)]
            cmr = -plsc.cummax(-jnp.flip(v, 0))
            smv = jnp.minimum(jnp.flip(cmr, 0), carry2)
            sm_v[pl.ds(16 * k, 16)] = smv
            carry2 = jnp.min(smv)

        pltpu.sync_copy(ctab_h.at[pl.ds(b * (L * CT) + l * CT, P)], c_v)
        accx = jnp.zeros((16,), jnp.float32)
        for k in range(P // 16):
            cvec = c_v[pl.ds(16 * k, 16)]
            below = pm_v[pl.ds(16 * k, 16)]
            above = plsc.load_gather(sm_v, [iota + (16 * k + 1)])
            d1 = cvec - below
            d2 = above - cvec
            dm = jnp.minimum(jnp.minimum(d1 * d1, d2 * d2),
                             jnp.float32(BIGD))
            accx = accx + dm
        cham_x = jnp.sum(accx) * jnp.float32(1.0 / P)

        accy = jnp.zeros((16,), jnp.float32)
        accc = jnp.zeros((16,), jnp.float32)
        for s in range(SPB):
            row = (b * SPB + s) * SROW
            pltpu.sync_copy(sums_h.at[pl.ds(row + l * 16, 16)], t16_v)
            accy = accy + t16_v[...]
            pltpu.sync_copy(sums_h.at[pl.ds(row + 4 * 16, 16)], t16_v)
            accc = accc + t16_v[...]
        cham_y_v = (jnp.full((16,), jnp.sum(accy), jnp.float32)
                    / jnp.full((16,), jnp.sum(accc), jnp.float32))

        out_v[...] = jnp.full((16,), cham_x, jnp.float32) + cham_y_v
        pltpu.sync_copy(out_v, out_h.at[pl.ds(w * 16, 16)])


def kernel(bins, target_depth_maps):
    assert bins.shape == (L, B, E)
    assert target_depth_maps.shape[0] == B
    assert target_depth_maps.size == B * M
    bins_f = bins.reshape(-1).astype(jnp.float32)
    y_f = target_depth_maps.reshape(-1).astype(jnp.float32)

    mesh = plsc.VectorSubcoreMesh(core_axis_name="c", subcore_axis_name="s")

    cparams = pltpu.CompilerParams(needs_layout_passes=False)

    k1 = functools.partial(
        pl.kernel, mesh=mesh,
        compiler_params=cparams,
        out_type=[
            jax.ShapeDtypeStruct((NW * IROW,), jnp.float32),
            jax.ShapeDtypeStruct((NW * IROW,), jnp.float32),
            jax.ShapeDtypeStruct((B * L * CT,), jnp.float32),
            jax.ShapeDtypeStruct((NW * SROW,), jnp.float32),
        ],
        scratch_types=[
            pltpu.VMEM((L * B * E,), jnp.float32),
            pltpu.VMEM((L * CT,), jnp.float32),
            pltpu.VMEM((MS,), jnp.float32),
            pltpu.VMEM((16,), jnp.float32),
            pltpu.VMEM((16,), jnp.int32),
            pltpu.VMEM((IROW,), jnp.float32),
            pltpu.VMEM((IROW,), jnp.float32),
            pltpu.VMEM((SROW,), jnp.float32),
        ],
    )(_k1_body)
    imax, imin, ctab, sums = k1(bins_f, y_f)

    k2 = functools.partial(
        pl.kernel, mesh=mesh,
        compiler_params=cparams,
        out_type=jax.ShapeDtypeStruct((L * B * 16,), jnp.float32),
        scratch_types=[
            pltpu.VMEM((RST,), jnp.float32),
            pltpu.VMEM((RST,), jnp.float32),
            pltpu.VMEM((RST,), jnp.float32),
            pltpu.VMEM((RST,), jnp.float32),
            pltpu.VMEM((RST,), jnp.float32),
            pltpu.VMEM((P,), jnp.float32),
            pltpu.VMEM((16,), jnp.float32),
            pltpu.VMEM((16,), jnp.float32),
        ],
    )(_k2_body)
    out = k2(imax, imin, ctab, sums)

    vals = out.reshape(L * B, 16)[:, 0]
    return jnp.sum(vals) / jnp.float32(B)


# speedup vs baseline: 1.1541x; 1.0510x over previous
import functools

import jax
import jax.numpy as jnp
from jax import lax
from jax.experimental import pallas as pl
from jax.experimental.pallas import tpu as pltpu
from jax.experimental.pallas import tpu_sc as plsc

L = 4
B = 4
P = 128
E = P + 1
M = 49152
NW = 32
SPB = 8
MS = M // SPB
U = 4
NI = MS // (16 * U)
CT = 256
RST = 144
IROW = L * RST
SROW = 128
BIGD = 1e10


def _cmpx(a, b):
    return jnp.minimum(a, b), jnp.maximum(a, b)


def _bclean(xs):
    if len(xs) == 1:
        return [jnp.sort(xs[0])]
    h = len(xs) // 2
    los, his = [], []
    for i in range(h):
        lo, hi = _cmpx(xs[i], xs[i + h])
        los.append(lo)
        his.append(hi)
    return _bclean(los) + _bclean(his)


def _msort(vs):
    if len(vs) == 1:
        return [jnp.sort(vs[0])]
    h = len(vs) // 2
    a = _msort(vs[:h])
    b = _msort(vs[h:])
    return _bclean(a + [jnp.flip(v, 0) for v in reversed(b)])


def _k1_body(bins_hbm, y_hbm, imax_o, imin_o, ctab_o, sums_o,
             bins_v, ctab_v, y_v, ys_v, imax_v, imin_v, sums_v):
    cid = lax.axis_index("c")
    sid = lax.axis_index("s")
    w = sid * 2 + cid
    b = w // SPB
    sl = w % SPB

    pltpu.sync_copy(bins_hbm, bins_v)
    pltpu.sync_copy(y_hbm.at[pl.ds(b * M + sl * MS, MS)], y_v)

    iota = lax.iota(jnp.int32, 16)
    big16 = jnp.full((16,), 1e30, jnp.float32)
    for k in range(L * CT // 16):
        ctab_v[pl.ds(16 * k, 16)] = big16

    for l in range(L):
        base = l * (B * E) + b * E
        vs = []
        for k in range(8):
            idx = iota + (base + 16 * k)
            e0 = plsc.load_gather(bins_v, [idx])
            e1 = plsc.load_gather(bins_v, [idx + 1])
            vs.append(0.5 * (e0 + e1))
        svs = _msort(vs)
        for k in range(8):
            ctab_v[pl.ds(l * CT + 16 * k, 16)] = svs[k]

    neg16 = jnp.full((16,), -2e5, jnp.float32)
    pos16 = jnp.full((16,), 2e5, jnp.float32)
    for k in range(U * IROW // 16):
        imax_v[pl.ds(16 * k, 16)] = neg16
        imin_v[pl.ds(16 * k, 16)] = pos16

    idx_m1 = jnp.maximum(iota - 1, 0)
    idx_p1 = jnp.minimum(iota + 1, 15)
    lane0 = iota == 0
    lane15 = iota == 15

    def body(i, carry):
        s0, s1, s2, s3, cntv = carry
        lsum = [s0, s1, s2, s3]
        for u in range(U):
            yv = y_v[pl.ds(i * (16 * U) + u * 16, 16)]
            yso = jnp.sort(yv)
            valid = yso >= 0.001
            cntv = cntv + jnp.where(valid, 1.0, 0.0)
            ys_v[pl.ds(u * 16, 16)] = yso
            prev_y = plsc.load_gather(ys_v, [idx_m1 + u * 16])
            next_y = plsc.load_gather(ys_v, [idx_p1 + u * 16])
            prev_valid = prev_y >= 0.001
            for l in range(L):
                cbase = l * CT
                pos0 = jnp.full((16,), cbase - 1, jnp.int32)
                for step in (128, 64, 32, 16, 8, 4, 2, 1):
                    cand = pos0 + step
                    cv = plsc.load_gather(ctab_v, [cand])
                    pos0 = jnp.where(cv <= yso, cand, pos0)
                rank = pos0 - (cbase - 1)
                lo = plsc.load_gather(ctab_v, [jnp.maximum(pos0, cbase)])
                hi = plsc.load_gather(ctab_v, [pos0 + 1])
                dlo = yso - lo
                dhi = hi - yso
                dmin = jnp.minimum(dlo * dlo, dhi * dhi)
                lsum[l] = lsum[l] + jnp.where(valid, dmin, 0.0)
                is_last = valid & (lane15 | (hi <= next_y))
                is_first = valid & (lane0 | ((pos0 >= cbase) & (lo > prev_y))
                                    | jnp.logical_not(prev_valid))
                ridx = rank + (l * RST + u * IROW)
                curmax = plsc.load_gather(imax_v, [ridx])
                plsc.store_scatter(imax_v, [ridx], jnp.maximum(curmax, yso),
                                   mask=is_last)
                curmin = plsc.load_gather(imin_v, [ridx])
                plsc.store_scatter(imin_v, [ridx], jnp.minimum(curmin, yso),
                                   mask=is_first)
        return (*lsum, cntv)

    z16 = jnp.zeros((16,), jnp.float32)
    s0, s1, s2, s3, cntv = lax.fori_loop(
        0, NI, body, (z16, z16, z16, z16, z16))

    for k in range(IROW // 16):
        mx = imax_v[pl.ds(16 * k, 16)]
        mn = imin_v[pl.ds(16 * k, 16)]
        for u in range(1, U):
            mx = jnp.maximum(mx, imax_v[pl.ds(u * IROW + 16 * k, 16)])
            mn = jnp.minimum(mn, imin_v[pl.ds(u * IROW + 16 * k, 16)])
        imax_v[pl.ds(16 * k, 16)] = mx
        imin_v[pl.ds(16 * k, 16)] = mn

    for l, sv in enumerate((s0, s1, s2, s3)):
        sums_v[pl.ds(l * 16, 16)] = sv
    sums_v[pl.ds(4 * 16, 16)] = cntv
    for k in range(5, 8):
        sums_v[pl.ds(k * 16, 16)] = z16

    pltpu.sync_copy(imax_v.at[pl.ds(0, IROW)], imax_o.at[pl.ds(w * IROW, IROW)])
    pltpu.sync_copy(imin_v.at[pl.ds(0, IROW)], imin_o.at[pl.ds(w * IROW, IROW)])
    pltpu.sync_copy(sums_v, sums_o.at[pl.ds(w * SROW, SROW)])

    @pl.when(sl == 0)
    def _():
        pltpu.sync_copy(ctab_v, ctab_o.at[pl.ds(b * (L * CT), L * CT)])


def _k2_body(imax_h, imin_h, ctab_h, sums_h, out_h,
             bmax_v, bmin_v, bsum_v, pm_v, sm_v, c_v, out_v):
    cid = lax.axis_index("c")
    sid = lax.axis_index("s")
    w = sid * 2 + cid

    @pl.when(w < L * B)
    def _():
        l = w // B
        b = w % B
        iota = lax.iota(jnp.int32, 16)
        base = l * RST

        pltpu.sync_copy(imax_h.at[pl.ds(b * SPB * IROW, SPB * IROW)], bmax_v)
        pltpu.sync_copy(imin_h.at[pl.ds(b * SPB * IROW, SPB * IROW)], bmin_v)
        pltpu.sync_copy(sums_h.at[pl.ds(b * SPB * SROW, SPB * SROW)], bsum_v)
        pltpu.sync_copy(ctab_h.at[pl.ds(b * (L * CT) + l * CT, P)], c_v)

        carry = jnp.float32(-3e5)
        for k in range(RST // 16):
            v = bmax_v[pl.ds(base + 16 * k, 16)]
            for s in range(1, SPB):
                v = jnp.maximum(v, bmax_v[pl.ds(s * IROW + base + 16 * k, 16)])
            cm = jnp.maximum(plsc.cummax(v), carry)
            pm_v[pl.ds(16 * k, 16)] = cm
            carry = jnp.max(cm)
        carry2 = jnp.float32(3e5)
        for k in range(RST // 16 - 1, -1, -1):
            v = bmin_v[pl.ds(base + 16 * k, 16)]
            for s in range(1, SPB):
                v = jnp.minimum(v, bmin_v[pl.ds(s * IROW + base + 16 * k, 16)])
            cmr = -plsc.cummax(-jnp.flip(v, 0))
            smv = jnp.minimum(jnp.flip(cmr, 0), carry2)
            sm_v[pl.ds(16 * k, 16)] = smv
            carry2 = jnp.min(smv)

        accx = jnp.zeros((16,), jnp.float32)
        for k in range(P // 16):
            cvec = c_v[pl.ds(16 * k, 16)]
            below = pm_v[pl.ds(16 * k, 16)]
            above = plsc.load_gather(sm_v, [iota + (16 * k + 1)])
            d1 = cvec - below
            d2 = above - cvec
            dm = jnp.minimum(jnp.minimum(d1 * d1, d2 * d2),
                             jnp.float32(BIGD))
            accx = accx + dm
        cham_x = jnp.sum(accx) * jnp.float32(1.0 / P)

        accy = jnp.zeros((16,), jnp.float32)
        accc = jnp.zeros((16,), jnp.float32)
        for s in range(SPB):
            row = s * SROW
            accy = accy + bsum_v[pl.ds(row + l * 16, 16)]
            accc = accc + bsum_v[pl.ds(row + 4 * 16, 16)]
        cham_y_v = (jnp.full((16,), jnp.sum(accy), jnp.float32)
                    / jnp.full((16,), jnp.sum(accc), jnp.float32))

        out_v[...] = jnp.full((16,), cham_x, jnp.float32) + cham_y_v
        pltpu.sync_copy(out_v, out_h.at[pl.ds(w * 16, 16)])


def kernel(bins, target_depth_maps):
    assert bins.shape == (L, B, E)
    assert target_depth_maps.shape[0] == B
    assert target_depth_maps.size == B * M
    bins_f = bins.reshape(-1).astype(jnp.float32)
    y_f = target_depth_maps.reshape(-1).astype(jnp.float32)

    mesh = plsc.VectorSubcoreMesh(core_axis_name="c", subcore_axis_name="s")
    cparams = pltpu.CompilerParams(needs_layout_passes=False)

    k1 = functools.partial(
        pl.kernel, mesh=mesh,
        compiler_params=cparams,
        out_type=[
            jax.ShapeDtypeStruct((NW * IROW,), jnp.float32),
            jax.ShapeDtypeStruct((NW * IROW,), jnp.float32),
            jax.ShapeDtypeStruct((B * L * CT,), jnp.float32),
            jax.ShapeDtypeStruct((NW * SROW,), jnp.float32),
        ],
        scratch_types=[
            pltpu.VMEM((L * B * E,), jnp.float32),
            pltpu.VMEM((L * CT,), jnp.float32),
            pltpu.VMEM((MS,), jnp.float32),
            pltpu.VMEM((U * 16,), jnp.float32),
            pltpu.VMEM((U * IROW,), jnp.float32),
            pltpu.VMEM((U * IROW,), jnp.float32),
            pltpu.VMEM((SROW,), jnp.float32),
        ],
    )(_k1_body)
    imax, imin, ctab, sums = k1(bins_f, y_f)

    k2 = functools.partial(
        pl.kernel, mesh=mesh,
        compiler_params=cparams,
        out_type=jax.ShapeDtypeStruct((L * B * 16,), jnp.float32),
        scratch_types=[
            pltpu.VMEM((SPB * IROW,), jnp.float32),
            pltpu.VMEM((SPB * IROW,), jnp.float32),
            pltpu.VMEM((SPB * SROW,), jnp.float32),
            pltpu.VMEM((RST,), jnp.float32),
            pltpu.VMEM((RST,), jnp.float32),
            pltpu.VMEM((P,), jnp.float32),
            pltpu.VMEM((16,), jnp.float32),
        ],
    )(_k2_body)
    out = k2(imax, imin, ctab, sums)

    vals = out.reshape(L * B, 16)[:, 0]
    return jnp.sum(vals) / jnp.float32(B)


# speedup vs baseline: 2.1446x; 1.8582x over previous
import functools

import jax
import jax.numpy as jnp
from jax import lax
from jax.experimental import pallas as pl
from jax.experimental.pallas import tpu as pltpu
from jax.experimental.pallas import tpu_sc as plsc

L = 4
B = 4
P = 128
E = P + 1
M = 49152
NW = 32
SPB = 8
MS = M // SPB
U = 4
NI = MS // (16 * U)
CT = 256
RST = 144
IROW = L * RST
SROW = 128
BIGD = 1e10


def _cmpx(a, b):
    return jnp.minimum(a, b), jnp.maximum(a, b)


def _bclean(xs):
    if len(xs) == 1:
        return [jnp.sort(xs[0])]
    h = len(xs) // 2
    los, his = [], []
    for i in range(h):
        lo, hi = _cmpx(xs[i], xs[i + h])
        los.append(lo)
        his.append(hi)
    return _bclean(los) + _bclean(his)


def _msort(vs):
    if len(vs) == 1:
        return [jnp.sort(vs[0])]
    h = len(vs) // 2
    a = _msort(vs[:h])
    b = _msort(vs[h:])
    return _bclean(a + [jnp.flip(v, 0) for v in reversed(b)])


def _k1_body(bins_hbm, y_hbm, imax_o, imin_o, ctab_o, sums_o,
             bins_v, ctab_v, y_v, *rest):
    ys_refs = list(rest[0:U])
    imax_refs = [list(rest[U + l * U:U + (l + 1) * U]) for l in range(L)]
    off = U + L * U
    imin_refs = [list(rest[off + l * U:off + (l + 1) * U]) for l in range(L)]
    sums_v = rest[off + L * U]

    cid = lax.axis_index("c")
    sid = lax.axis_index("s")
    w = sid * 2 + cid
    b = w // SPB
    sl = w % SPB

    pltpu.sync_copy(bins_hbm, bins_v)
    pltpu.sync_copy(y_hbm.at[pl.ds(b * M + sl * MS, MS)], y_v)

    iota = lax.iota(jnp.int32, 16)
    big16 = jnp.full((16,), 1e30, jnp.float32)
    for k in range(L * CT // 16):
        ctab_v[pl.ds(16 * k, 16)] = big16

    for l in range(L):
        base = l * (B * E) + b * E
        vs = []
        for k in range(8):
            idx = iota + (base + 16 * k)
            e0 = plsc.load_gather(bins_v, [idx])
            e1 = plsc.load_gather(bins_v, [idx + 1])
            vs.append(0.5 * (e0 + e1))
        svs = _msort(vs)
        for k in range(8):
            ctab_v[pl.ds(l * CT + 16 * k, 16)] = svs[k]

    neg16 = jnp.full((16,), -2e5, jnp.float32)
    pos16 = jnp.full((16,), 2e5, jnp.float32)
    for l in range(L):
        for u in range(U):
            for k in range(RST // 16):
                imax_refs[l][u][pl.ds(16 * k, 16)] = neg16
                imin_refs[l][u][pl.ds(16 * k, 16)] = pos16

    idx_m1 = jnp.maximum(iota - 1, 0)
    idx_p1 = jnp.minimum(iota + 1, 15)
    lane0 = iota == 0
    lane15 = iota == 15
    UL = [(u, l) for u in range(U) for l in range(L)]

    def body(i, carry):
        s0, s1, s2, s3, cntv = carry
        lsum = [s0, s1, s2, s3]

        yso, valid, prev_y, next_y = [], [], [], []
        for u in range(U):
            yv = y_v[pl.ds(i * (16 * U) + u * 16, 16)]
            yso.append(jnp.sort(yv))
            valid.append(yso[u] >= 0.001)
        for u in range(U):
            cntv = cntv + jnp.where(valid[u], 1.0, 0.0)
            ys_refs[u][...] = yso[u]
        for u in range(U):
            prev_y.append(plsc.load_gather(ys_refs[u], [idx_m1]))
            next_y.append(plsc.load_gather(ys_refs[u], [idx_p1]))

        pos = {(u, l): jnp.full((16,), l * CT - 1, jnp.int32) for u, l in UL}
        for step in (128, 64, 32, 16, 8, 4, 2, 1):
            cand = {}
            cv = {}
            for u, l in UL:
                cand[u, l] = pos[u, l] + step
                cv[u, l] = plsc.load_gather(ctab_v, [cand[u, l]])
            for u, l in UL:
                pos[u, l] = jnp.where(cv[u, l] <= yso[u], cand[u, l],
                                      pos[u, l])

        lo = {}
        hi = {}
        for u, l in UL:
            lo[u, l] = plsc.load_gather(
                ctab_v, [jnp.maximum(pos[u, l], l * CT)])
            hi[u, l] = plsc.load_gather(ctab_v, [pos[u, l] + 1])
        rank = {}
        for u, l in UL:
            dlo = yso[u] - lo[u, l]
            dhi = hi[u, l] - yso[u]
            dmin = jnp.minimum(dlo * dlo, dhi * dhi)
            lsum[l] = lsum[l] + jnp.where(valid[u], dmin, 0.0)
            rank[u, l] = pos[u, l] - (l * CT - 1)

        curmax = {}
        curmin = {}
        for u, l in UL:
            curmax[u, l] = plsc.load_gather(imax_refs[l][u], [rank[u, l]])
            curmin[u, l] = plsc.load_gather(imin_refs[l][u], [rank[u, l]])
        for u, l in UL:
            is_last = valid[u] & (lane15 | (hi[u, l] <= next_y[u]))
            is_first = valid[u] & (lane0
                                   | ((pos[u, l] >= l * CT)
                                      & (lo[u, l] > prev_y[u]))
                                   | jnp.logical_not(prev_y[u] >= 0.001))
            plsc.store_scatter(imax_refs[l][u], [rank[u, l]],
                               jnp.maximum(curmax[u, l], yso[u]),
                               mask=is_last)
            plsc.store_scatter(imin_refs[l][u], [rank[u, l]],
                               jnp.minimum(curmin[u, l], yso[u]),
                               mask=is_first)
        return (*lsum, cntv)

    z16 = jnp.zeros((16,), jnp.float32)
    s0, s1, s2, s3, cntv = lax.fori_loop(
        0, NI, body, (z16, z16, z16, z16, z16))

    for l in range(L):
        for k in range(RST // 16):
            mx = imax_refs[l][0][pl.ds(16 * k, 16)]
            mn = imin_refs[l][0][pl.ds(16 * k, 16)]
            for u in range(1, U):
                mx = jnp.maximum(mx, imax_refs[l][u][pl.ds(16 * k, 16)])
                mn = jnp.minimum(mn, imin_refs[l][u][pl.ds(16 * k, 16)])
            imax_refs[l][0][pl.ds(16 * k, 16)] = mx
            imin_refs[l][0][pl.ds(16 * k, 16)] = mn
        pltpu.sync_copy(imax_refs[l][0],
                        imax_o.at[pl.ds(w * IROW + l * RST, RST)])
        pltpu.sync_copy(imin_refs[l][0],
                        imin_o.at[pl.ds(w * IROW + l * RST, RST)])

    for l, sv in enumerate((s0, s1, s2, s3)):
        sums_v[pl.ds(l * 16, 16)] = sv
    sums_v[pl.ds(4 * 16, 16)] = cntv
    for k in range(5, 8):
        sums_v[pl.ds(k * 16, 16)] = z16
    pltpu.sync_copy(sums_v, sums_o.at[pl.ds(w * SROW, SROW)])

    @pl.when(sl == 0)
    def _():
        pltpu.sync_copy(ctab_v, ctab_o.at[pl.ds(b * (L * CT), L * CT)])


def _k2_body(imax_h, imin_h, ctab_h, sums_h, out_h,
             bmax_v, bmin_v, bsum_v, pm_v, sm_v, c_v, out_v):
    cid = lax.axis_index("c")
    sid = lax.axis_index("s")
    w = sid * 2 + cid

    @pl.when(w < L * B)
    def _():
        l = w // B
        b = w % B
        iota = lax.iota(jnp.int32, 16)
        base = l * RST

        pltpu.sync_copy(imax_h.at[pl.ds(b * SPB * IROW, SPB * IROW)], bmax_v)
        pltpu.sync_copy(imin_h.at[pl.ds(b * SPB * IROW, SPB * IROW)], bmin_v)
        pltpu.sync_copy(sums_h.at[pl.ds(b * SPB * SROW, SPB * SROW)], bsum_v)
        pltpu.sync_copy(ctab_h.at[pl.ds(b * (L * CT) + l * CT, P)], c_v)

        carry = jnp.float32(-3e5)
        for k in range(RST // 16):
            v = bmax_v[pl.ds(base + 16 * k, 16)]
            for s in range(1, SPB):
                v = jnp.maximum(v, bmax_v[pl.ds(s * IROW + base + 16 * k, 16)])
            cm = jnp.maximum(plsc.cummax(v), carry)
            pm_v[pl.ds(16 * k, 16)] = cm
            carry = jnp.max(cm)
        carry2 = jnp.float32(3e5)
        for k in range(RST // 16 - 1, -1, -1):
            v = bmin_v[pl.ds(base + 16 * k, 16)]
            for s in range(1, SPB):
                v = jnp.minimum(v, bmin_v[pl.ds(s * IROW + base + 16 * k, 16)])
            cmr = -plsc.cummax(-jnp.flip(v, 0))
            smv = jnp.minimum(jnp.flip(cmr, 0), carry2)
            sm_v[pl.ds(16 * k, 16)] = smv
            carry2 = jnp.min(smv)

        accx = jnp.zeros((16,), jnp.float32)
        for k in range(P // 16):
            cvec = c_v[pl.ds(16 * k, 16)]
            below = pm_v[pl.ds(16 * k, 16)]
            above = plsc.load_gather(sm_v, [iota + (16 * k + 1)])
            d1 = cvec - below
            d2 = above - cvec
            dm = jnp.minimum(jnp.minimum(d1 * d1, d2 * d2),
                             jnp.float32(BIGD))
            accx = accx + dm
        cham_x = jnp.sum(accx) * jnp.float32(1.0 / P)

        accy = jnp.zeros((16,), jnp.float32)
        accc = jnp.zeros((16,), jnp.float32)
        for s in range(SPB):
            row = s * SROW
            accy = accy + bsum_v[pl.ds(row + l * 16, 16)]
            accc = accc + bsum_v[pl.ds(row + 4 * 16, 16)]
        cham_y_v = (jnp.full((16,), jnp.sum(accy), jnp.float32)
                    / jnp.full((16,), jnp.sum(accc), jnp.float32))

        out_v[...] = jnp.full((16,), cham_x, jnp.float32) + cham_y_v
        pltpu.sync_copy(out_v, out_h.at[pl.ds(w * 16, 16)])


def kernel(bins, target_depth_maps):
    assert bins.shape == (L, B, E)
    assert target_depth_maps.shape[0] == B
    assert target_depth_maps.size == B * M
    bins_f = bins.reshape(-1).astype(jnp.float32)
    y_f = target_depth_maps.reshape(-1).astype(jnp.float32)

    mesh = plsc.VectorSubcoreMesh(core_axis_name="c", subcore_axis_name="s")
    cparams = pltpu.CompilerParams(needs_layout_passes=False)

    k1_scratch = [
        pltpu.VMEM((L * B * E,), jnp.float32),
        pltpu.VMEM((L * CT,), jnp.float32),
        pltpu.VMEM((MS,), jnp.float32),
    ]
    k1_scratch += [pltpu.VMEM((16,), jnp.float32) for _ in range(U)]
    k1_scratch += [pltpu.VMEM((RST,), jnp.float32) for _ in range(L * U)]
    k1_scratch += [pltpu.VMEM((RST,), jnp.float32) for _ in range(L * U)]
    k1_scratch += [pltpu.VMEM((SROW,), jnp.float32)]

    k1 = functools.partial(
        pl.kernel, mesh=mesh,
        compiler_params=cparams,
        out_type=[
            jax.ShapeDtypeStruct((NW * IROW,), jnp.float32),
            jax.ShapeDtypeStruct((NW * IROW,), jnp.float32),
            jax.ShapeDtypeStruct((B * L * CT,), jnp.float32),
            jax.ShapeDtypeStruct((NW * SROW,), jnp.float32),
        ],
        scratch_types=k1_scratch,
    )(_k1_body)
    imax, imin, ctab, sums = k1(bins_f, y_f)

    k2 = functools.partial(
        pl.kernel, mesh=mesh,
        compiler_params=cparams,
        out_type=jax.ShapeDtypeStruct((L * B * 16,), jnp.float32),
        scratch_types=[
            pltpu.VMEM((SPB * IROW,), jnp.float32),
            pltpu.VMEM((SPB * IROW,), jnp.float32),
            pltpu.VMEM((SPB * SROW,), jnp.float32),
            pltpu.VMEM((RST,), jnp.float32),
            pltpu.VMEM((RST,), jnp.float32),
            pltpu.VMEM((P,), jnp.float32),
            pltpu.VMEM((16,), jnp.float32),
        ],
    )(_k2_body)
    out = k2(imax, imin, ctab, sums)

    vals = out.reshape(L * B, 16)[:, 0]
    return jnp.sum(vals) / jnp.float32(B)


# speedup vs baseline: 3.0622x; 1.4279x over previous
import functools

import jax
import jax.numpy as jnp
from jax import lax
from jax.experimental import pallas as pl
from jax.experimental.pallas import tpu as pltpu
from jax.experimental.pallas import tpu_sc as plsc

L = 4
B = 4
P = 128
E = P + 1
M = 49152
NW = 32
SPB = 8
MS = M // SPB
U = 4
NI = MS // (16 * U)
CT = 256
MT = 1024
CNT = 528
RST = 144
IROW = L * RST
SROW = 128
BIGD = 1e10


def _cmpx(a, b):
    return jnp.minimum(a, b), jnp.maximum(a, b)


def _bclean(xs):
    if len(xs) == 1:
        return [jnp.sort(xs[0])]
    h = len(xs) // 2
    los, his = [], []
    for i in range(h):
        lo, hi = _cmpx(xs[i], xs[i + h])
        los.append(lo)
        his.append(hi)
    return _bclean(los) + _bclean(his)


def _msort(vs):
    if len(vs) == 1:
        return [jnp.sort(vs[0])]
    h = len(vs) // 2
    a = _msort(vs[:h])
    b = _msort(vs[h:])
    return _bclean(a + [jnp.flip(v, 0) for v in reversed(b)])


def _cmpx_kv(a, b):
    m = a[0] <= b[0]
    lo = (jnp.minimum(a[0], b[0]), jnp.where(m, a[1], b[1]))
    hi = (jnp.maximum(a[0], b[0]), jnp.where(m, b[1], a[1]))
    return lo, hi


def _bclean_kv(xs):
    if len(xs) == 1:
        k, v = plsc.sort_key_val(xs[0][0], xs[0][1])
        return [(k, v)]
    h = len(xs) // 2
    los, his = [], []
    for i in range(h):
        lo, hi = _cmpx_kv(xs[i], xs[i + h])
        los.append(lo)
        his.append(hi)
    return _bclean_kv(los) + _bclean_kv(his)


def _merge_kv(a, b):
    rb = [(jnp.flip(k, 0), jnp.flip(v, 0)) for k, v in reversed(b)]
    return _bclean_kv(a + rb)


def _k1_body(bins_hbm, y_hbm, imax_o, imin_o, ctab_o, sums_o,
             bins_v, ctab_v, mt_v, *rest):
    cnt_refs = list(rest[0:L])
    y_v = rest[L]
    ys_refs = list(rest[L + 1:L + 1 + U])
    off = L + 1 + U
    imax_refs = [list(rest[off + l * U:off + (l + 1) * U]) for l in range(L)]
    off += L * U
    imin_refs = [list(rest[off + l * U:off + (l + 1) * U]) for l in range(L)]
    sums_v = rest[off + L * U]

    cid = lax.axis_index("c")
    sid = lax.axis_index("s")
    w = sid * 2 + cid
    b = w // SPB
    sl = w % SPB

    pltpu.sync_copy(bins_hbm, bins_v)
    pltpu.sync_copy(y_hbm.at[pl.ds(b * M + sl * MS, MS)], y_v)

    iota = lax.iota(jnp.int32, 16)
    big16 = jnp.full((16,), 1e30, jnp.float32)
    for k in range(L * CT // 16):
        ctab_v[pl.ds(16 * k, 16)] = big16

    per_level = []
    for l in range(L):
        base = l * (B * E) + b * E
        vs = []
        for k in range(8):
            idx = iota + (base + 16 * k)
            e0 = plsc.load_gather(bins_v, [idx])
            e1 = plsc.load_gather(bins_v, [idx + 1])
            vs.append(0.5 * (e0 + e1))
        svs = _msort(vs)
        per_level.append(svs)
        for k in range(8):
            ctab_v[pl.ds(l * CT + 16 * k, 16)] = svs[k]

    kv = []
    for l in range(L):
        tag = jnp.full((16,), l, jnp.int32)
        kv.append([(v, tag) for v in per_level[l]])
    m01 = _merge_kv(kv[0], kv[1])
    m23 = _merge_kv(kv[2], kv[3])
    mall = _merge_kv(m01, m23)
    for k in range(MT // 16):
        mt_v[pl.ds(16 * k, 16)] = big16
    for k in range(32):
        mt_v[pl.ds(16 * k, 16)] = mall[k][0]
    z16i = jnp.zeros((16,), jnp.int32)
    for l in range(L):
        cnt_refs[l][pl.ds(0, 16)] = z16i
        carry_c = jnp.int32(0)
        for k in range(32):
            ind = jnp.where(mall[k][1] == l, 1, 0).astype(jnp.int32)
            incl = plsc.cumsum(ind) + carry_c
            plsc.store_scatter(cnt_refs[l], [iota + (16 * k + 1)], incl)
            carry_c = carry_c + jnp.sum(ind)

    neg16 = jnp.full((16,), -2e5, jnp.float32)
    pos16 = jnp.full((16,), 2e5, jnp.float32)
    for l in range(L):
        for u in range(U):
            for k in range(RST // 16):
                imax_refs[l][u][pl.ds(16 * k, 16)] = neg16
                imin_refs[l][u][pl.ds(16 * k, 16)] = pos16

    idx_m1 = jnp.maximum(iota - 1, 0)
    idx_p1 = jnp.minimum(iota + 1, 15)
    lane0 = iota == 0
    lane15 = iota == 15

    def body(i, carry):
        s0, s1, s2, s3, cntv = carry
        lsum = [s0, s1, s2, s3]

        yso, valid, prev_y, next_y = [], [], [], []
        for u in range(U):
            yv = y_v[pl.ds(i * (16 * U) + u * 16, 16)]
            yso.append(jnp.sort(yv))
            valid.append(yso[u] >= 0.001)
        for u in range(U):
            cntv = cntv + jnp.where(valid[u], 1.0, 0.0)
            ys_refs[u][...] = yso[u]
        for u in range(U):
            prev_y.append(plsc.load_gather(ys_refs[u], [idx_m1]))
            next_y.append(plsc.load_gather(ys_refs[u], [idx_p1]))

        pos = [jnp.full((16,), -1, jnp.int32) for _ in range(U)]
        for step in (512, 256, 128, 64, 32, 16, 8, 4, 2, 1):
            cand = []
            cv = []
            for u in range(U):
                cand.append(pos[u] + step)
                cv.append(plsc.load_gather(mt_v, [cand[u]]))
            for u in range(U):
                pos[u] = jnp.where(cv[u] <= yso[u], cand[u], pos[u])
        rmerged = [pos[u] + 1 for u in range(U)]

        not_pv = [jnp.logical_not(prev_y[u] >= 0.001) for u in range(U)]
        for l in range(L):
            rank = []
            lo = []
            hi = []
            for u in range(U):
                rank.append(plsc.load_gather(cnt_refs[l], [rmerged[u]]))
            for u in range(U):
                lo.append(plsc.load_gather(
                    ctab_v, [jnp.maximum(rank[u] - 1, 0) + l * CT]))
                hi.append(plsc.load_gather(
                    ctab_v, [rank[u] + l * CT]))
            curmax = []
            curmin = []
            for u in range(U):
                curmax.append(plsc.load_gather(imax_refs[l][u], [rank[u]]))
                curmin.append(plsc.load_gather(imin_refs[l][u], [rank[u]]))
            for u in range(U):
                dlo = yso[u] - lo[u]
                dhi = hi[u] - yso[u]
                dmin = jnp.minimum(dlo * dlo, dhi * dhi)
                lsum[l] = lsum[l] + jnp.where(valid[u], dmin, 0.0)
                is_last = valid[u] & (lane15 | (hi[u] <= next_y[u]))
                is_first = valid[u] & (lane0
                                       | ((rank[u] >= 1) & (lo[u] > prev_y[u]))
                                       | not_pv[u])
                plsc.store_scatter(imax_refs[l][u], [rank[u]],
                                   jnp.maximum(curmax[u], yso[u]),
                                   mask=is_last)
                plsc.store_scatter(imin_refs[l][u], [rank[u]],
                                   jnp.minimum(curmin[u], yso[u]),
                                   mask=is_first)
        return (*lsum, cntv)

    z16 = jnp.zeros((16,), jnp.float32)
    s0, s1, s2, s3, cntv = lax.fori_loop(
        0, NI, body, (z16, z16, z16, z16, z16))

    for l in range(L):
        for k in range(RST // 16):
            mx = imax_refs[l][0][pl.ds(16 * k, 16)]
            mn = imin_refs[l][0][pl.ds(16 * k, 16)]
            for u in range(1, U):
                mx = jnp.maximum(mx, imax_refs[l][u][pl.ds(16 * k, 16)])
                mn = jnp.minimum(mn, imin_refs[l][u][pl.ds(16 * k, 16)])
            imax_refs[l][0][pl.ds(16 * k, 16)] = mx
            imin_refs[l][0][pl.ds(16 * k, 16)] = mn
        pltpu.sync_copy(imax_refs[l][0],
                        imax_o.at[pl.ds(w * IROW + l * RST, RST)])
        pltpu.sync_copy(imin_refs[l][0],
                        imin_o.at[pl.ds(w * IROW + l * RST, RST)])

    for l, sv in enumerate((s0, s1, s2, s3)):
        sums_v[pl.ds(l * 16, 16)] = sv
    sums_v[pl.ds(4 * 16, 16)] = cntv
    for k in range(5, 8):
        sums_v[pl.ds(k * 16, 16)] = z16
    pltpu.sync_copy(sums_v, sums_o.at[pl.ds(w * SROW, SROW)])

    @pl.when(sl == 0)
    def _():
        pltpu.sync_copy(ctab_v, ctab_o.at[pl.ds(b * (L * CT), L * CT)])


def _k2_body(imax_h, imin_h, ctab_h, sums_h, out_h,
             bmax_v, bmin_v, bsum_v, pm_v, sm_v, c_v, out_v):
    cid = lax.axis_index("c")
    sid = lax.axis_index("s")
    w = sid * 2 + cid

    @pl.when(w < L * B)
    def _():
        l = w // B
        b = w % B
        iota = lax.iota(jnp.int32, 16)
        base = l * RST

        pltpu.sync_copy(imax_h.at[pl.ds(b * SPB * IROW, SPB * IROW)], bmax_v)
        pltpu.sync_copy(imin_h.at[pl.ds(b * SPB * IROW, SPB * IROW)], bmin_v)
        pltpu.sync_copy(sums_h.at[pl.ds(b * SPB * SROW, SPB * SROW)], bsum_v)
        pltpu.sync_copy(ctab_h.at[pl.ds(b * (L * CT) + l * CT, P)], c_v)

        carry = jnp.float32(-3e5)
        for k in range(RST // 16):
            v = bmax_v[pl.ds(base + 16 * k, 16)]
            for s in range(1, SPB):
                v = jnp.maximum(v, bmax_v[pl.ds(s * IROW + base + 16 * k, 16)])
            cm = jnp.maximum(plsc.cummax(v), carry)
            pm_v[pl.ds(16 * k, 16)] = cm
            carry = jnp.max(cm)
        carry2 = jnp.float32(3e5)
        for k in range(RST // 16 - 1, -1, -1):
            v = bmin_v[pl.ds(base + 16 * k, 16)]
            for s in range(1, SPB):
                v = jnp.minimum(v, bmin_v[pl.ds(s * IROW + base + 16 * k, 16)])
            cmr = -plsc.cummax(-jnp.flip(v, 0))
            smv = jnp.minimum(jnp.flip(cmr, 0), carry2)
            sm_v[pl.ds(16 * k, 16)] = smv
            carry2 = jnp.min(smv)

        accx = jnp.zeros((16,), jnp.float32)
        for k in range(P // 16):
            cvec = c_v[pl.ds(16 * k, 16)]
            below = pm_v[pl.ds(16 * k, 16)]
            above = plsc.load_gather(sm_v, [iota + (16 * k + 1)])
            d1 = cvec - below
            d2 = above - cvec
            dm = jnp.minimum(jnp.minimum(d1 * d1, d2 * d2),
                             jnp.float32(BIGD))
            accx = accx + dm
        cham_x = jnp.sum(accx) * jnp.float32(1.0 / P)

        accy = jnp.zeros((16,), jnp.float32)
        accc = jnp.zeros((16,), jnp.float32)
        for s in range(SPB):
            row = s * SROW
            accy = accy + bsum_v[pl.ds(row + l * 16, 16)]
            accc = accc + bsum_v[pl.ds(row + 4 * 16, 16)]
        cham_y_v = (jnp.full((16,), jnp.sum(accy), jnp.float32)
                    / jnp.full((16,), jnp.sum(accc), jnp.float32))

        out_v[...] = jnp.full((16,), cham_x, jnp.float32) + cham_y_v
        pltpu.sync_copy(out_v, out_h.at[pl.ds(w * 16, 16)])


def kernel(bins, target_depth_maps):
    assert bins.shape == (L, B, E)
    assert target_depth_maps.shape[0] == B
    assert target_depth_maps.size == B * M
    bins_f = bins.reshape(-1).astype(jnp.float32)
    y_f = target_depth_maps.reshape(-1).astype(jnp.float32)

    mesh = plsc.VectorSubcoreMesh(core_axis_name="c", subcore_axis_name="s")
    cparams = pltpu.CompilerParams(needs_layout_passes=False)

    k1_scratch = [
        pltpu.VMEM((L * B * E,), jnp.float32),
        pltpu.VMEM((L * CT,), jnp.float32),
        pltpu.VMEM((MT,), jnp.float32),
    ]
    k1_scratch += [pltpu.VMEM((CNT,), jnp.int32) for _ in range(L)]
    k1_scratch += [pltpu.VMEM((MS,), jnp.float32)]
    k1_scratch += [pltpu.VMEM((16,), jnp.float32) for _ in range(U)]
    k1_scratch += [pltpu.VMEM((RST,), jnp.float32) for _ in range(L * U)]
    k1_scratch += [pltpu.VMEM((RST,), jnp.float32) for _ in range(L * U)]
    k1_scratch += [pltpu.VMEM((SROW,), jnp.float32)]

    k1 = functools.partial(
        pl.kernel, mesh=mesh,
        compiler_params=cparams,
        out_type=[
            jax.ShapeDtypeStruct((NW * IROW,), jnp.float32),
            jax.ShapeDtypeStruct((NW * IROW,), jnp.float32),
            jax.ShapeDtypeStruct((B * L * CT,), jnp.float32),
            jax.ShapeDtypeStruct((NW * SROW,), jnp.float32),
        ],
        scratch_types=k1_scratch,
    )(_k1_body)
    imax, imin, ctab, sums = k1(bins_f, y_f)

    k2 = functools.partial(
        pl.kernel, mesh=mesh,
        compiler_params=cparams,
        out_type=jax.ShapeDtypeStruct((L * B * 16,), jnp.float32),
        scratch_types=[
            pltpu.VMEM((SPB * IROW,), jnp.float32),
            pltpu.VMEM((SPB * IROW,), jnp.float32),
            pltpu.VMEM((SPB * SROW,), jnp.float32),
            pltpu.VMEM((RST,), jnp.float32),
            pltpu.VMEM((RST,), jnp.float32),
            pltpu.VMEM((P,), jnp.float32),
            pltpu.VMEM((16,), jnp.float32),
        ],
    )(_k2_body)
    out = k2(imax, imin, ctab, sums)

    vals = out.reshape(L * B, 16)[:, 0]
    return jnp.sum(vals) / jnp.float32(B)


# speedup vs baseline: 3.2213x; 1.0520x over previous
import functools

import jax
import jax.numpy as jnp
from jax import lax
from jax.experimental import pallas as pl
from jax.experimental.pallas import tpu as pltpu
from jax.experimental.pallas import tpu_sc as plsc

L = 4
B = 4
P = 128
E = P + 1
M = 49152
NW = 32
SPB = 8
MS = M // SPB
U = 4
NI = MS // (16 * U)
CT = 256
MT = 1024
CNT = 528
RST = 144
IROW = L * RST
SROW = 128
BIGD = 1e10


def _cmpx(a, b):
    return jnp.minimum(a, b), jnp.maximum(a, b)


def _bclean(xs):
    if len(xs) == 1:
        return [jnp.sort(xs[0])]
    h = len(xs) // 2
    los, his = [], []
    for i in range(h):
        lo, hi = _cmpx(xs[i], xs[i + h])
        los.append(lo)
        his.append(hi)
    return _bclean(los) + _bclean(his)


def _msort(vs):
    if len(vs) == 1:
        return [jnp.sort(vs[0])]
    h = len(vs) // 2
    a = _msort(vs[:h])
    b = _msort(vs[h:])
    return _bclean(a + [jnp.flip(v, 0) for v in reversed(b)])


def _cmpx_kv(a, b):
    m = a[0] <= b[0]
    lo = (jnp.minimum(a[0], b[0]), jnp.where(m, a[1], b[1]))
    hi = (jnp.maximum(a[0], b[0]), jnp.where(m, b[1], a[1]))
    return lo, hi


def _bclean_kv(xs):
    if len(xs) == 1:
        k, v = plsc.sort_key_val(xs[0][0], xs[0][1])
        return [(k, v)]
    h = len(xs) // 2
    los, his = [], []
    for i in range(h):
        lo, hi = _cmpx_kv(xs[i], xs[i + h])
        los.append(lo)
        his.append(hi)
    return _bclean_kv(los) + _bclean_kv(his)


def _merge_kv(a, b):
    rb = [(jnp.flip(k, 0), jnp.flip(v, 0)) for k, v in reversed(b)]
    return _bclean_kv(a + rb)


def _k1_body(bins_hbm, y_hbm, out_h,
             bins_v, ctab_v, mt_v, *rest):
    cnt_refs = list(rest[0:L])
    y_v = rest[L]
    ys_refs = list(rest[L + 1:L + 1 + U])
    off = L + 1 + U
    imax_refs = [list(rest[off + l * U:off + (l + 1) * U]) for l in range(L)]
    off += L * U
    imin_refs = [list(rest[off + l * U:off + (l + 1) * U]) for l in range(L)]
    (sums_v, imax_s, imin_s, sums_s, ctab_s,
     bmax_v, bmin_v, bsum_v, pm_v, sm_v, c_v, out_v) = rest[off + L * U:]

    cid = lax.axis_index("c")
    sid = lax.axis_index("s")
    w = cid * 16 + sid
    b = w // SPB
    sl = w % SPB

    pltpu.sync_copy(bins_hbm, bins_v)
    pltpu.sync_copy(y_hbm.at[pl.ds(b * M + sl * MS, MS)], y_v)

    iota = lax.iota(jnp.int32, 16)
    big16 = jnp.full((16,), 1e30, jnp.float32)
    for k in range(L * CT // 16):
        ctab_v[pl.ds(16 * k, 16)] = big16

    per_level = []
    for l in range(L):
        base = l * (B * E) + b * E
        vs = []
        for k in range(8):
            idx = iota + (base + 16 * k)
            e0 = plsc.load_gather(bins_v, [idx])
            e1 = plsc.load_gather(bins_v, [idx + 1])
            vs.append(0.5 * (e0 + e1))
        svs = _msort(vs)
        per_level.append(svs)
        for k in range(8):
            ctab_v[pl.ds(l * CT + 16 * k, 16)] = svs[k]

    kv = []
    for l in range(L):
        tag = jnp.full((16,), l, jnp.int32)
        kv.append([(v, tag) for v in per_level[l]])
    m01 = _merge_kv(kv[0], kv[1])
    m23 = _merge_kv(kv[2], kv[3])
    mall = _merge_kv(m01, m23)
    for k in range(MT // 16):
        mt_v[pl.ds(16 * k, 16)] = big16
    for k in range(32):
        mt_v[pl.ds(16 * k, 16)] = mall[k][0]
    z16i = jnp.zeros((16,), jnp.int32)
    for l in range(L):
        cnt_refs[l][pl.ds(0, 16)] = z16i
        carry_c = jnp.int32(0)
        for k in range(32):
            ind = jnp.where(mall[k][1] == l, 1, 0).astype(jnp.int32)
            incl = plsc.cumsum(ind) + carry_c
            plsc.store_scatter(cnt_refs[l], [iota + (16 * k + 1)], incl)
            carry_c = carry_c + jnp.sum(ind)

    neg16 = jnp.full((16,), -2e5, jnp.float32)
    pos16 = jnp.full((16,), 2e5, jnp.float32)
    for l in range(L):
        for u in range(U):
            for k in range(RST // 16):
                imax_refs[l][u][pl.ds(16 * k, 16)] = neg16
                imin_refs[l][u][pl.ds(16 * k, 16)] = pos16

    idx_m1 = jnp.maximum(iota - 1, 0)
    idx_p1 = jnp.minimum(iota + 1, 15)
    lane0 = iota == 0
    lane15 = iota == 15

    def body(i, carry):
        s0, s1, s2, s3, cntv = carry
        lsum = [s0, s1, s2, s3]

        yso, valid, prev_y, next_y = [], [], [], []
        for u in range(U):
            yv = y_v[pl.ds(i * (16 * U) + u * 16, 16)]
            yso.append(jnp.sort(yv))
            valid.append(yso[u] >= 0.001)
        for u in range(U):
            cntv = cntv + jnp.where(valid[u], 1.0, 0.0)
            ys_refs[u][...] = yso[u]
        for u in range(U):
            prev_y.append(plsc.load_gather(ys_refs[u], [idx_m1]))
            next_y.append(plsc.load_gather(ys_refs[u], [idx_p1]))

        pos = [jnp.full((16,), -1, jnp.int32) for _ in range(U)]
        for step in (512, 256, 128, 64, 32, 16, 8, 4, 2, 1):
            cand = []
            cv = []
            for u in range(U):
                cand.append(pos[u] + step)
                cv.append(plsc.load_gather(mt_v, [cand[u]]))
            for u in range(U):
                pos[u] = jnp.where(cv[u] <= yso[u], cand[u], pos[u])
        rmerged = [pos[u] + 1 for u in range(U)]

        not_pv = [jnp.logical_not(prev_y[u] >= 0.001) for u in range(U)]
        for l in range(L):
            rank = []
            lo = []
            hi = []
            for u in range(U):
                rank.append(plsc.load_gather(cnt_refs[l], [rmerged[u]]))
            for u in range(U):
                lo.append(plsc.load_gather(
                    ctab_v, [jnp.maximum(rank[u] - 1, 0) + l * CT]))
                hi.append(plsc.load_gather(
                    ctab_v, [rank[u] + l * CT]))
            curmax = []
            curmin = []
            for u in range(U):
                curmax.append(plsc.load_gather(imax_refs[l][u], [rank[u]]))
                curmin.append(plsc.load_gather(imin_refs[l][u], [rank[u]]))
            for u in range(U):
                dlo = yso[u] - lo[u]
                dhi = hi[u] - yso[u]
                dmin = jnp.minimum(dlo * dlo, dhi * dhi)
                lsum[l] = lsum[l] + jnp.where(valid[u], dmin, 0.0)
                is_last = valid[u] & (lane15 | (hi[u] <= next_y[u]))
                is_first = valid[u] & (lane0
                                       | ((rank[u] >= 1) & (lo[u] > prev_y[u]))
                                       | not_pv[u])
                plsc.store_scatter(imax_refs[l][u], [rank[u]],
                                   jnp.maximum(curmax[u], yso[u]),
                                   mask=is_last)
                plsc.store_scatter(imin_refs[l][u], [rank[u]],
                                   jnp.minimum(curmin[u], yso[u]),
                                   mask=is_first)
        return (*lsum, cntv)

    z16 = jnp.zeros((16,), jnp.float32)
    s0, s1, s2, s3, cntv = lax.fori_loop(
        0, NI, body, (z16, z16, z16, z16, z16))

    for l in range(L):
        for k in range(RST // 16):
            mx = imax_refs[l][0][pl.ds(16 * k, 16)]
            mn = imin_refs[l][0][pl.ds(16 * k, 16)]
            for u in range(1, U):
                mx = jnp.maximum(mx, imax_refs[l][u][pl.ds(16 * k, 16)])
                mn = jnp.minimum(mn, imin_refs[l][u][pl.ds(16 * k, 16)])
            imax_refs[l][0][pl.ds(16 * k, 16)] = mx
            imin_refs[l][0][pl.ds(16 * k, 16)] = mn
        pltpu.sync_copy(imax_refs[l][0],
                        imax_s.at[pl.ds(sid * IROW + l * RST, RST)])
        pltpu.sync_copy(imin_refs[l][0],
                        imin_s.at[pl.ds(sid * IROW + l * RST, RST)])

    for l, sv in enumerate((s0, s1, s2, s3)):
        sums_v[pl.ds(l * 16, 16)] = sv
    sums_v[pl.ds(4 * 16, 16)] = cntv
    for k in range(5, 8):
        sums_v[pl.ds(k * 16, 16)] = z16
    pltpu.sync_copy(sums_v, sums_s.at[pl.ds(sid * SROW, SROW)])

    @pl.when(sl == 0)
    def _():
        pltpu.sync_copy(ctab_v, ctab_s.at[pl.ds((sid // SPB) * (L * CT),
                                                L * CT)])

    plsc.subcore_barrier()

    @pl.when(sid < SPB)
    def _():
        l = sid % L
        lb = sid // L
        base = l * RST
        iota2 = lax.iota(jnp.int32, 16)

        pltpu.sync_copy(imax_s.at[pl.ds(lb * SPB * IROW, SPB * IROW)], bmax_v)
        pltpu.sync_copy(imin_s.at[pl.ds(lb * SPB * IROW, SPB * IROW)], bmin_v)
        pltpu.sync_copy(sums_s.at[pl.ds(lb * SPB * SROW, SPB * SROW)], bsum_v)
        pltpu.sync_copy(ctab_s.at[pl.ds(lb * (L * CT) + l * CT, P)], c_v)

        carry = jnp.float32(-3e5)
        for k in range(RST // 16):
            v = bmax_v[pl.ds(base + 16 * k, 16)]
            for s in range(1, SPB):
                v = jnp.maximum(v, bmax_v[pl.ds(s * IROW + base + 16 * k, 16)])
            cm = jnp.maximum(plsc.cummax(v), carry)
            pm_v[pl.ds(16 * k, 16)] = cm
            carry = jnp.max(cm)
        carry2 = jnp.float32(3e5)
        for k in range(RST // 16 - 1, -1, -1):
            v = bmin_v[pl.ds(base + 16 * k, 16)]
            for s in range(1, SPB):
                v = jnp.minimum(v, bmin_v[pl.ds(s * IROW + base + 16 * k, 16)])
            cmr = -plsc.cummax(-jnp.flip(v, 0))
            smv = jnp.minimum(jnp.flip(cmr, 0), carry2)
            sm_v[pl.ds(16 * k, 16)] = smv
            carry2 = jnp.min(smv)

        accx = jnp.zeros((16,), jnp.float32)
        for k in range(P // 16):
            cvec = c_v[pl.ds(16 * k, 16)]
            below = pm_v[pl.ds(16 * k, 16)]
            above = plsc.load_gather(sm_v, [iota2 + (16 * k + 1)])
            d1 = cvec - below
            d2 = above - cvec
            dm = jnp.minimum(jnp.minimum(d1 * d1, d2 * d2),
                             jnp.float32(BIGD))
            accx = accx + dm
        cham_x = jnp.sum(accx) * jnp.float32(1.0 / P)

        accy = jnp.zeros((16,), jnp.float32)
        accc = jnp.zeros((16,), jnp.float32)
        for s in range(SPB):
            row = s * SROW
            accy = accy + bsum_v[pl.ds(row + l * 16, 16)]
            accc = accc + bsum_v[pl.ds(row + 4 * 16, 16)]
        cham_y_v = (jnp.full((16,), jnp.sum(accy), jnp.float32)
                    / jnp.full((16,), jnp.sum(accc), jnp.float32))

        out_v[...] = jnp.full((16,), cham_x, jnp.float32) + cham_y_v
        gb = cid * 2 + lb
        pltpu.sync_copy(out_v, out_h.at[pl.ds((l * B + gb) * 16, 16)])


def kernel(bins, target_depth_maps):
    assert bins.shape == (L, B, E)
    assert target_depth_maps.shape[0] == B
    assert target_depth_maps.size == B * M
    bins_f = bins.reshape(-1).astype(jnp.float32)
    y_f = target_depth_maps.reshape(-1).astype(jnp.float32)

    mesh = plsc.VectorSubcoreMesh(core_axis_name="c", subcore_axis_name="s")
    cparams = pltpu.CompilerParams(needs_layout_passes=False)

    k1_scratch = [
        pltpu.VMEM((L * B * E,), jnp.float32),
        pltpu.VMEM((L * CT,), jnp.float32),
        pltpu.VMEM((MT,), jnp.float32),
    ]
    k1_scratch += [pltpu.VMEM((CNT,), jnp.int32) for _ in range(L)]
    k1_scratch += [pltpu.VMEM((MS,), jnp.float32)]
    k1_scratch += [pltpu.VMEM((16,), jnp.float32) for _ in range(U)]
    k1_scratch += [pltpu.VMEM((RST,), jnp.float32) for _ in range(L * U)]
    k1_scratch += [pltpu.VMEM((RST,), jnp.float32) for _ in range(L * U)]
    k1_scratch += [
        pltpu.VMEM((SROW,), jnp.float32),
        pltpu.VMEM_SHARED((16 * IROW,), jnp.float32),
        pltpu.VMEM_SHARED((16 * IROW,), jnp.float32),
        pltpu.VMEM_SHARED((16 * SROW,), jnp.float32),
        pltpu.VMEM_SHARED((2 * L * CT,), jnp.float32),
        pltpu.VMEM((SPB * IROW,), jnp.float32),
        pltpu.VMEM((SPB * IROW,), jnp.float32),
        pltpu.VMEM((SPB * SROW,), jnp.float32),
        pltpu.VMEM((RST,), jnp.float32),
        pltpu.VMEM((RST,), jnp.float32),
        pltpu.VMEM((P,), jnp.float32),
        pltpu.VMEM((16,), jnp.float32),
    ]

    k1 = functools.partial(
        pl.kernel, mesh=mesh,
        compiler_params=cparams,
        out_type=jax.ShapeDtypeStruct((L * B * 16,), jnp.float32),
        scratch_types=k1_scratch,
    )(_k1_body)
    out = k1(bins_f, y_f)

    vals = out.reshape(L * B, 16)[:, 0]
    return jnp.sum(vals) / jnp.float32(B)
